# baseline (speedup 1.0000x reference)
import functools

import jax
import jax.numpy as jnp
from jax import lax
from jax.experimental import pallas as pl
from jax.experimental.pallas import tpu as pltpu
from jax.experimental.pallas import tpu_sc as plsc

_NC = 2
_NS = 16
_NW = _NC * _NS
_L = 16
_D = 1024
_BATCH = 4
_SEQ = 2048
_ROWS_W = _SEQ // _NW
_CH = 16
_CH_EL = _CH * _D
_PE_EL = _ROWS_W * _D
_CHUNKS_PER_BATCH = _ROWS_W // _CH
_T = _BATCH * _CHUNKS_PER_BATCH


def _sc_body(x_hbm, pe_hbm, o_hbm, pe_v, x_bufs,
             in_sem0, in_sem1, out_sem0, out_sem1):
    cid = lax.axis_index("c")
    sid = lax.axis_index("s")
    wid = sid * _NC + cid
    pe_base = wid * _PE_EL
    pltpu.sync_copy(pe_hbm.at[pl.ds(pe_base, _PE_EL)], pe_v)

    in_sems = (in_sem0, in_sem1)
    out_sems = (out_sem0, out_sem1)

    def row0(t):
        b = t // _CHUNKS_PER_BATCH
        c = t % _CHUNKS_PER_BATCH
        return b * (_SEQ * _D) + pe_base + c * _CH_EL

    in_d = [None, None]
    out_d = [None, None]

    def start_in(t, s):
        in_d[s] = pltpu.async_copy(
            x_hbm.at[pl.ds(row0(t), _CH_EL)], x_bufs.at[s], in_sems[s])

    def compute(s, t):
        c = t % _CHUNKS_PER_BATCH
        pe_off = c * _CH_EL

        @plsc.parallel_loop(0, _CH_EL // _L, unroll=16)
        def _(i):
            sl = pl.ds(i * _L, _L)
            x_bufs[s, sl] = x_bufs[s, sl] + pe_v[pl.ds(pe_off + i * _L, _L)]

    start_in(0, 0)
    for t in range(_T):
        s = t % 2
        if t + 1 < _T:
            if t >= 1:
                out_d[1 - s].wait()
            start_in(t + 1, 1 - s)
        in_d[s].wait()
        compute(s, t)
        out_d[s] = pltpu.async_copy(
            x_bufs.at[s], o_hbm.at[pl.ds(row0(t), _CH_EL)], out_sems[s])
    out_d[0].wait()
    out_d[1].wait()


_sc_kernel = functools.partial(
    pl.kernel,
    out_type=jax.ShapeDtypeStruct((_BATCH * _SEQ * _D,), jnp.float32),
    mesh=plsc.VectorSubcoreMesh(core_axis_name="c", subcore_axis_name="s"),
    scratch_types=[
        pltpu.VMEM((_PE_EL,), jnp.float32),
        pltpu.VMEM((2, _CH_EL), jnp.float32),
        pltpu.SemaphoreType.DMA,
        pltpu.SemaphoreType.DMA,
        pltpu.SemaphoreType.DMA,
        pltpu.SemaphoreType.DMA,
    ],
)(_sc_body)


def kernel(x, pe_weight):
    B, S, D = x.shape
    out = _sc_kernel(x.reshape(-1), pe_weight.reshape(-1))
    return out.reshape(B, S, D)


# speedup vs baseline: 2.8713x; 2.8713x over previous
import functools

import jax
import jax.numpy as jnp
from jax import lax
from jax.experimental import pallas as pl
from jax.experimental.pallas import tpu as pltpu
from jax.experimental.pallas import tpu_sc as plsc

_NC = 2
_NS = 16
_NW = _NC * _NS
_L = 16
_D = 1024
_BATCH = 4
_SEQ = 2048
_ROWS_W = _SEQ // _NW
_PASS_ROWS = 32
_N_PASS = _ROWS_W // _PASS_ROWS
_CH = 16
_CPB = _PASS_ROWS // _CH
_T = _BATCH * _CPB
_NBUF = 4
_SLICES = _CH * (_D // _L)


def _sc_body(x_hbm, pe_hbm, o_hbm, pe_v, x_bufs,
             is0, is1, is2, is3, os0, os1, os2, os3):
    cid = lax.axis_index("c")
    sid = lax.axis_index("s")
    wid = sid * _NC + cid
    band0 = wid * _ROWS_W
    in_sems = (is0, is1, is2, is3)
    out_sems = (os0, os1, os2, os3)

    def pass_body(p, carry):
        pe_row0 = band0 + p * _PASS_ROWS

        def xrow(t):
            b = t // _CPB
            c = t % _CPB
            return b * _SEQ + pe_row0 + c * _CH

        def start_in(t):
            s = t % _NBUF
            return pltpu.async_copy(
                x_hbm.at[pl.ds(xrow(t), _CH)], x_bufs.at[s], in_sems[s])

        in_d = {}
        out_d = {}
        in_d[0] = start_in(0)
        in_d[1] = start_in(1)
        pltpu.sync_copy(pe_hbm.at[pl.ds(pe_row0, _PASS_ROWS)], pe_v)

        for t in range(_T):
            s = t % _NBUF
            if t + 2 < _T:
                if t >= 2:
                    out_d[t - 2].wait()
                in_d[t + 2] = start_in(t + 2)
            in_d[t].wait()
            c = t % _CPB

            @plsc.parallel_loop(0, _SLICES, unroll=8)
            def _(i):
                r = i >> 6
                sl = pl.ds((i & 63) * _L, _L)
                v = pe_v[c * _CH + r, sl]
                plsc.addupdate(x_bufs.at[s, r, sl], v)

            out_d[t] = pltpu.async_copy(
                x_bufs.at[s], o_hbm.at[pl.ds(xrow(t), _CH)], out_sems[s])
        for t in range(_T - _NBUF, _T):
            out_d[t].wait()
        return carry

    lax.fori_loop(0, _N_PASS, pass_body, 0)


_sc_kernel = functools.partial(
    pl.kernel,
    out_type=jax.ShapeDtypeStruct((_BATCH * _SEQ, _D), jnp.float32),
    mesh=plsc.VectorSubcoreMesh(core_axis_name="c", subcore_axis_name="s"),
    scratch_types=[
        pltpu.VMEM((_PASS_ROWS, _D), jnp.float32),
        pltpu.VMEM((_NBUF, _CH, _D), jnp.float32),
        pltpu.SemaphoreType.DMA,
        pltpu.SemaphoreType.DMA,
        pltpu.SemaphoreType.DMA,
        pltpu.SemaphoreType.DMA,
        pltpu.SemaphoreType.DMA,
        pltpu.SemaphoreType.DMA,
        pltpu.SemaphoreType.DMA,
        pltpu.SemaphoreType.DMA,
    ],
)(_sc_body)


def kernel(x, pe_weight):
    B, S, D = x.shape
    out = _sc_kernel(x.reshape(B * S, D), pe_weight)
    return out.reshape(B, S, D)
